# baseline (speedup 1.0000x reference)
import functools

import jax
import jax.numpy as jnp
from jax import lax
from jax.experimental import pallas as pl
from jax.experimental.pallas import tpu as pltpu
from jax.experimental.pallas import tpu_sc as plsc

_NT = 16
_GC = 64


def _sc_stage(ei, ew, w_pad, attr_pad, table):
    npad = w_pad.shape[0]
    n_edges = ei.shape[1]
    nchunks_e = n_edges // 128
    base_ch = nchunks_e // _NT
    n_extra = nchunks_e - base_ch * _NT
    ept = (base_ch + 1) * 128
    rpt = npad // _NT
    f32 = jnp.float32

    mesh = plsc.VectorSubcoreMesh(core_axis_name="c", subcore_axis_name="s")

    def body(ei_hbm, ew_hbm, w_hbm, attr_hbm, table_hbm,
             s_hbm, t_hbm, emb_hbm,
             eb2_v, wb_v, g_v, acc_v, red_v,
             wc_v, cc_v, rc_v, gc_v,
             idx_v, rows_v, rows2_v, rows3_v,
             parts_sh, gfull_sh, zeros_sh,
             sem_g0, sem_g1, sem_g2, sem_o0, sem_o1, sem_o2,
             sem_e0, sem_e1):
        cid = lax.axis_index("c")
        tid = lax.axis_index("s")

        @pl.when(cid == 0)
        def _edge_passes():
            main_e = base_ch * 128
            ebase = tid * main_e + 128 * jnp.minimum(tid, n_extra)
            has_extra = tid < n_extra
            nbase = tid * rpt
            pltpu.sync_copy(ei_hbm.at[:, pl.ds(ebase, main_e)],
                            eb2_v.at[:, pl.ds(0, main_e)])
            h_ew = pltpu.async_copy(ew_hbm.at[pl.ds(ebase, main_e)],
                                    wb_v.at[pl.ds(0, main_e)], sem_e1)

            @pl.when(has_extra)
            def _extra_loads():
                pltpu.sync_copy(ei_hbm.at[:, pl.ds(ebase + main_e, 128)],
                                eb2_v.at[:, pl.ds(main_e, 128)])
                pltpu.sync_copy(ew_hbm.at[pl.ds(ebase + main_e, 128)],
                                wb_v.at[pl.ds(main_e, 128)])
            pltpu.sync_copy(w_hbm.at[pl.ds(nbase, rpt)], wc_v)

            zero16 = jnp.zeros((16,), f32)
            ones16 = jnp.ones((16,), f32)

            def zero_acc():
                def zb(i, c):
                    acc_v[pl.ds(i * 16, 16)] = zero16
                    return c
                lax.fori_loop(0, npad // 16, zb, 0)

            def reduce_to(chunk_ref):
                pltpu.sync_copy(acc_v, parts_sh.at[tid])
                plsc.subcore_barrier()
                pltpu.sync_copy(parts_sh.at[:, pl.ds(nbase, rpt)], red_v)
                plsc.subcore_barrier()

                def rb(j, c):
                    sl0 = pl.ds(j * 32, 16)
                    sl1 = pl.ds(j * 32 + 16, 16)
                    v0 = red_v[0, sl0]
                    v1 = red_v[0, sl1]
                    for k in range(1, _NT):
                        v0 = v0 + red_v[k, sl0]
                        v1 = v1 + red_v[k, sl1]
                    chunk_ref[sl0] = v0
                    chunk_ref[sl1] = v1
                    return c
                lax.fori_loop(0, rpt // 32, rb, 0)

            def share_g(num_ref):
                def gb(j, c):
                    sl = pl.ds(j * 16, 16)
                    gc_v[sl] = num_ref[sl] / jnp.maximum(cc_v[sl], 1.0)
                    return c
                lax.fori_loop(0, rpt // 16, gb, 0)
                pltpu.sync_copy(gc_v, gfull_sh.at[pl.ds(nbase, rpt)])
                plsc.subcore_barrier()
                pltpu.sync_copy(gfull_sh, g_v)

            zero_acc()

            @pl.when(tid == 0)
            def _stash_zeros():
                pltpu.sync_copy(acc_v, zeros_sh)

            unroll = 8

            def p0(i, c):
                sls = [pl.ds((i * unroll + u) * 16, 16) for u in range(unroll)]
                ds_ = [eb2_v[1, sl] for sl in sls]
                for d in ds_:
                    plsc.addupdate_scatter(acc_v, [d], ones16)
                return c
            lax.fori_loop(0, base_ch, p0, 0)

            @pl.when(has_extra)
            def _p0_extra():
                p0(base_ch, 0)
            h_ew.wait()
            reduce_to(cc_v)
            share_g(wc_v)

            def edge_pass(i, c):
                sls = [pl.ds((i * unroll + u) * 16, 16) for u in range(unroll)]
                ds_ = [eb2_v[1, sl] for sl in sls]
                gs_ = [plsc.load_gather(g_v, [d]) for d in ds_]
                es_ = [wb_v[sl] for sl in sls]
                ss_ = [eb2_v[0, sl] for sl in sls]
                vals = [g * e for g, e in zip(gs_, es_)]
                for s_, v in zip(ss_, vals):
                    plsc.addupdate_scatter(acc_v, [s_], v)
                return c

            def run_edge_pass():
                lax.fori_loop(0, base_ch, edge_pass, 0)

                @pl.when(has_extra)
                def _extra():
                    edge_pass(base_ch, 0)

            pltpu.sync_copy(zeros_sh, acc_v)
            run_edge_pass()
            reduce_to(rc_v)
            h_s = pltpu.async_copy(rc_v, s_hbm.at[pl.ds(nbase, rpt)], sem_e0)
            share_g(rc_v)

            pltpu.sync_copy(zeros_sh, acc_v)
            run_edge_pass()
            reduce_to(cc_v)
            pltpu.sync_copy(cc_v, t_hbm.at[pl.ds(nbase, rpt)])
            h_s.wait()

        @pl.when(cid == 1)
        def _emb_gather():
            r0 = tid * rpt
            nchunks = rpt // _GC
            pltpu.sync_copy(attr_hbm.at[pl.ds(r0, rpt)], idx_v)
            nbuf = 3
            rowsb = [rows_v, rows2_v, rows3_v]
            gsem = [sem_g0, sem_g1, sem_g2]
            osem = [sem_o0, sem_o1, sem_o2]
            gh = [None] * nbuf
            oh = [None] * nbuf
            for j in range(min(nbuf - 1, nchunks)):
                gh[j] = pltpu.async_copy(
                    table_hbm.at[idx_v.at[pl.ds(j * _GC, _GC)]],
                    rowsb[j], gsem[j])
            for j in range(nchunks):
                p = j % nbuf
                jn = j + nbuf - 1
                if jn < nchunks:
                    q = jn % nbuf
                    if oh[q] is not None:
                        oh[q].wait()
                    gh[q] = pltpu.async_copy(
                        table_hbm.at[idx_v.at[pl.ds(jn * _GC, _GC)]],
                        rowsb[q], gsem[q])
                gh[p].wait()
                oh[p] = pltpu.async_copy(
                    rowsb[p], emb_hbm.at[pl.ds(r0 + j * _GC, _GC), :], osem[p])
            for p in range(nbuf):
                if oh[p] is not None:
                    oh[p].wait()

    call = pl.kernel(
        body,
        mesh=mesh,
        compiler_params=pltpu.CompilerParams(needs_layout_passes=False),
        out_type=[
            jax.ShapeDtypeStruct((npad,), f32),
            jax.ShapeDtypeStruct((npad,), f32),
            jax.ShapeDtypeStruct((npad, 128), f32),
        ],
        scratch_types=[
            pltpu.VMEM((2, ept), jnp.int32),
            pltpu.VMEM((ept,), f32),
            pltpu.VMEM((npad,), f32),
            pltpu.VMEM((npad,), f32),
            pltpu.VMEM((_NT, rpt), f32),
            pltpu.VMEM((rpt,), f32),
            pltpu.VMEM((rpt,), f32),
            pltpu.VMEM((rpt,), f32),
            pltpu.VMEM((rpt,), f32),
            pltpu.VMEM((rpt,), jnp.int32),
            pltpu.VMEM((_GC, 128), f32),
            pltpu.VMEM((_GC, 128), f32),
            pltpu.VMEM((_GC, 128), f32),
            pltpu.VMEM_SHARED((_NT, npad), f32),
            pltpu.VMEM_SHARED((npad,), f32),
            pltpu.VMEM_SHARED((npad,), f32),
            pltpu.SemaphoreType.DMA,
            pltpu.SemaphoreType.DMA,
            pltpu.SemaphoreType.DMA,
            pltpu.SemaphoreType.DMA,
            pltpu.SemaphoreType.DMA,
            pltpu.SemaphoreType.DMA,
            pltpu.SemaphoreType.DMA,
            pltpu.SemaphoreType.DMA,
        ],
    )
    return call(ei, ew, w_pad, attr_pad, table)


def _tc_body(emb_ref, w_ref, s_ref, t_ref, ws1_ref, wn1_ref, b1_ref,
             ws2_ref, wn2_ref, b2_ref, out_ref):
    hi = jax.lax.Precision.HIGHEST

    def dot(a, b):
        return lax.dot_general(a, b, (((1,), (0,)), ((), ())),
                               precision=hi, preferred_element_type=jnp.float32)

    emb = emb_ref[:, :]
    ssq = jnp.sum(emb * emb, axis=1, keepdims=True)
    scale = jnp.minimum(1.0, lax.rsqrt(jnp.maximum(ssq, 1e-30)))
    m = emb * scale
    w = w_ref[:, :]
    s = s_ref[:, :]
    t = t_ref[:, :]
    q = jnp.concatenate([w * w, s * w, t * w], axis=0)
    abc = dot(q, m)
    a_, b_, c_ = abc[0:1], abc[1:2], abc[2:3]
    sw = jnp.sum(w)
    ss = jnp.sum(s)
    b1 = b1_ref[:, :]
    b2 = b2_ref[:, :]
    u = dot(a_, ws1_ref[:, :]) + sw * b1 + dot(b_, wn1_ref[:, :])
    v = dot(b_, ws1_ref[:, :]) + ss * b1 + dot(c_, wn1_ref[:, :])
    out_ref[:, :] = (dot(u, ws2_ref[:, :]) + sw * b2 + dot(v, wn2_ref[:, :])) / sw


def kernel(node_attr, node_w, edge_index, edge_w, embed_table,
           W_self1, W_neigh1, b1, W_self2, W_neigh2, b2):
    n = node_attr.shape[0]
    npad = ((n + 16 * _GC - 1) // (16 * _GC)) * (16 * _GC)
    pad = npad - n

    attr = node_attr.astype(jnp.int32)
    attr_pad = jnp.concatenate([attr, jnp.zeros((pad,), jnp.int32)])
    w = node_w.reshape(-1).astype(jnp.float32)
    w_pad = jnp.concatenate([w, jnp.zeros((pad,), jnp.float32)])
    ei = edge_index.astype(jnp.int32)
    ew = edge_w.astype(jnp.float32)

    s, t, emb_dense = _sc_stage(ei, ew, w_pad, attr_pad, embed_table)

    out = pl.pallas_call(
        _tc_body,
        out_shape=jax.ShapeDtypeStruct((1, 32), jnp.float32),
    )(emb_dense, w_pad.reshape(1, -1), s.reshape(1, -1), t.reshape(1, -1),
      W_self1, W_neigh1, b1.reshape(1, -1), W_self2, W_neigh2, b2.reshape(1, -1))
    return out


# speedup vs baseline: 1.0061x; 1.0061x over previous
import functools

import jax
import jax.numpy as jnp
from jax import lax
from jax.experimental import pallas as pl
from jax.experimental.pallas import tpu as pltpu
from jax.experimental.pallas import tpu_sc as plsc

_NT = 16
_GC = 64


def _sc_stage(ei, ew, w_pad, attr_pad, table):
    npad = w_pad.shape[0]
    n_edges = ei.shape[1]
    nchunks_e = n_edges // 128
    base_ch = nchunks_e // _NT
    n_extra = nchunks_e - base_ch * _NT
    ept = (base_ch + 1) * 128
    rpt = npad // _NT
    f32 = jnp.float32

    mesh = plsc.VectorSubcoreMesh(core_axis_name="c", subcore_axis_name="s")

    def body(ei_hbm, ew_hbm, w_hbm, attr_hbm, table_hbm,
             s_hbm, t_hbm, emb_hbm,
             eb2_v, wb_v, g_v, acc_v, red_v,
             wc_v, cc_v, rc_v, gc_v,
             idx_v, rows_v, rows2_v,
             parts_sh, gfull_sh, zeros_sh,
             sem_g0, sem_g1, sem_o0, sem_o1, sem_e0, sem_e1):
        cid = lax.axis_index("c")
        tid = lax.axis_index("s")

        @pl.when(cid == 0)
        def _edge_passes():
            main_e = base_ch * 128
            ebase = tid * main_e + 128 * jnp.minimum(tid, n_extra)
            has_extra = tid < n_extra
            nbase = tid * rpt
            pltpu.sync_copy(ei_hbm.at[:, pl.ds(ebase, main_e)],
                            eb2_v.at[:, pl.ds(0, main_e)])
            h_ew = pltpu.async_copy(ew_hbm.at[pl.ds(ebase, main_e)],
                                    wb_v.at[pl.ds(0, main_e)], sem_e1)

            @pl.when(has_extra)
            def _extra_loads():
                pltpu.sync_copy(ei_hbm.at[:, pl.ds(ebase + main_e, 128)],
                                eb2_v.at[:, pl.ds(main_e, 128)])
                pltpu.sync_copy(ew_hbm.at[pl.ds(ebase + main_e, 128)],
                                wb_v.at[pl.ds(main_e, 128)])
            pltpu.sync_copy(w_hbm.at[pl.ds(nbase, rpt)], wc_v)

            zero16 = jnp.zeros((16,), f32)
            ones16 = jnp.ones((16,), f32)

            def zero_acc():
                def zb(i, c):
                    acc_v[pl.ds(i * 16, 16)] = zero16
                    return c
                lax.fori_loop(0, npad // 16, zb, 0)

            def reduce_to(chunk_ref):
                pltpu.sync_copy(acc_v, parts_sh.at[tid])
                plsc.subcore_barrier()
                pltpu.sync_copy(parts_sh.at[:, pl.ds(nbase, rpt)], red_v)
                plsc.subcore_barrier()

                def rb(j, c):
                    sl0 = pl.ds(j * 32, 16)
                    sl1 = pl.ds(j * 32 + 16, 16)
                    v0 = red_v[0, sl0]
                    v1 = red_v[0, sl1]
                    for k in range(1, _NT):
                        v0 = v0 + red_v[k, sl0]
                        v1 = v1 + red_v[k, sl1]
                    chunk_ref[sl0] = v0
                    chunk_ref[sl1] = v1
                    return c
                lax.fori_loop(0, rpt // 32, rb, 0)

            def share_g(num_ref):
                def gb(j, c):
                    sl = pl.ds(j * 16, 16)
                    gc_v[sl] = num_ref[sl] / jnp.maximum(cc_v[sl], 1.0)
                    return c
                lax.fori_loop(0, rpt // 16, gb, 0)
                pltpu.sync_copy(gc_v, gfull_sh.at[pl.ds(nbase, rpt)])
                plsc.subcore_barrier()
                pltpu.sync_copy(gfull_sh, g_v)

            zero_acc()

            @pl.when(tid == 0)
            def _stash_zeros():
                pltpu.sync_copy(acc_v, zeros_sh)

            unroll = 8

            def p0(i, c):
                sls = [pl.ds((i * unroll + u) * 16, 16) for u in range(unroll)]
                ds_ = [eb2_v[1, sl] for sl in sls]
                for d in ds_:
                    plsc.addupdate_scatter(acc_v, [d], ones16)
                return c
            lax.fori_loop(0, base_ch, p0, 0)

            @pl.when(has_extra)
            def _p0_extra():
                p0(base_ch, 0)
            h_ew.wait()
            reduce_to(cc_v)
            share_g(wc_v)

            def edge_pass(i, c):
                sls = [pl.ds((i * unroll + u) * 16, 16) for u in range(unroll)]
                ds_ = [eb2_v[1, sl] for sl in sls]
                gs_ = [plsc.load_gather(g_v, [d]) for d in ds_]
                es_ = [wb_v[sl] for sl in sls]
                ss_ = [eb2_v[0, sl] for sl in sls]
                vals = [g * e for g, e in zip(gs_, es_)]
                for s_, v in zip(ss_, vals):
                    plsc.addupdate_scatter(acc_v, [s_], v)
                return c

            def run_edge_pass():
                lax.fori_loop(0, base_ch, edge_pass, 0)

                @pl.when(has_extra)
                def _extra():
                    edge_pass(base_ch, 0)

            pltpu.sync_copy(zeros_sh, acc_v)
            run_edge_pass()
            reduce_to(rc_v)
            pltpu.sync_copy(rc_v, s_hbm.at[pl.ds(nbase, rpt)])
            share_g(rc_v)

            pltpu.sync_copy(zeros_sh, acc_v)
            run_edge_pass()
            reduce_to(rc_v)
            pltpu.sync_copy(rc_v, t_hbm.at[pl.ds(nbase, rpt)])

        @pl.when(cid == 1)
        def _emb_gather():
            r0 = tid * rpt
            nchunks = rpt // _GC
            pltpu.sync_copy(attr_hbm.at[pl.ds(r0, rpt)], idx_v)
            rowsb = [rows_v, rows2_v]
            gsem = [sem_g0, sem_g1]
            osem = [sem_o0, sem_o1]
            gh = [None, None]
            oh = [None, None]
            gh[0] = pltpu.async_copy(
                table_hbm.at[idx_v.at[pl.ds(0, _GC)]], rows_v, sem_g0)
            for j in range(nchunks):
                p = j % 2
                q = 1 - p
                if j + 1 < nchunks:
                    if oh[q] is not None:
                        oh[q].wait()
                    gh[q] = pltpu.async_copy(
                        table_hbm.at[idx_v.at[pl.ds((j + 1) * _GC, _GC)]],
                        rowsb[q], gsem[q])
                gh[p].wait()
                oh[p] = pltpu.async_copy(
                    rowsb[p], emb_hbm.at[pl.ds(r0 + j * _GC, _GC), :], osem[p])
            for p in range(2):
                if oh[p] is not None:
                    oh[p].wait()

    call = pl.kernel(
        body,
        mesh=mesh,
        compiler_params=pltpu.CompilerParams(needs_layout_passes=False),
        out_type=[
            jax.ShapeDtypeStruct((npad,), f32),
            jax.ShapeDtypeStruct((npad,), f32),
            jax.ShapeDtypeStruct((npad, 128), f32),
        ],
        scratch_types=[
            pltpu.VMEM((2, ept), jnp.int32),
            pltpu.VMEM((ept,), f32),
            pltpu.VMEM((npad,), f32),
            pltpu.VMEM((npad,), f32),
            pltpu.VMEM((_NT, rpt), f32),
            pltpu.VMEM((rpt,), f32),
            pltpu.VMEM((rpt,), f32),
            pltpu.VMEM((rpt,), f32),
            pltpu.VMEM((rpt,), f32),
            pltpu.VMEM((rpt,), jnp.int32),
            pltpu.VMEM((_GC, 128), f32),
            pltpu.VMEM((_GC, 128), f32),
            pltpu.VMEM_SHARED((_NT, npad), f32),
            pltpu.VMEM_SHARED((npad,), f32),
            pltpu.VMEM_SHARED((npad,), f32),
            pltpu.SemaphoreType.DMA,
            pltpu.SemaphoreType.DMA,
            pltpu.SemaphoreType.DMA,
            pltpu.SemaphoreType.DMA,
            pltpu.SemaphoreType.DMA,
            pltpu.SemaphoreType.DMA,
        ],
    )
    return call(ei, ew, w_pad, attr_pad, table)


def _tc_body(emb_ref, w_ref, s_ref, t_ref, ws1_ref, wn1_ref, b1_ref,
             ws2_ref, wn2_ref, b2_ref, out_ref):
    hi = jax.lax.Precision.HIGHEST

    def dot(a, b):
        return lax.dot_general(a, b, (((1,), (0,)), ((), ())),
                               precision=hi, preferred_element_type=jnp.float32)

    emb = emb_ref[:, :]
    ssq = jnp.sum(emb * emb, axis=1, keepdims=True)
    scale = jnp.minimum(1.0, lax.rsqrt(jnp.maximum(ssq, 1e-30)))
    m = emb * scale
    w = w_ref[:, :]
    s = s_ref[:, :]
    t = t_ref[:, :]
    q = jnp.concatenate([w * w, s * w, t * w], axis=0)
    abc = dot(q, m)
    a_, b_, c_ = abc[0:1], abc[1:2], abc[2:3]
    sw = jnp.sum(w)
    ss = jnp.sum(s)
    b1 = b1_ref[:, :]
    b2 = b2_ref[:, :]
    u = dot(a_, ws1_ref[:, :]) + sw * b1 + dot(b_, wn1_ref[:, :])
    v = dot(b_, ws1_ref[:, :]) + ss * b1 + dot(c_, wn1_ref[:, :])
    out_ref[:, :] = (dot(u, ws2_ref[:, :]) + sw * b2 + dot(v, wn2_ref[:, :])) / sw


def kernel(node_attr, node_w, edge_index, edge_w, embed_table,
           W_self1, W_neigh1, b1, W_self2, W_neigh2, b2):
    n = node_attr.shape[0]
    npad = ((n + 16 * _GC - 1) // (16 * _GC)) * (16 * _GC)
    pad = npad - n

    attr = node_attr.astype(jnp.int32)
    attr_pad = jnp.concatenate([attr, jnp.zeros((pad,), jnp.int32)])
    w = node_w.reshape(-1).astype(jnp.float32)
    w_pad = jnp.concatenate([w, jnp.zeros((pad,), jnp.float32)])
    ei = edge_index.astype(jnp.int32)
    ew = edge_w.astype(jnp.float32)

    s, t, emb_dense = _sc_stage(ei, ew, w_pad, attr_pad, embed_table)

    out = pl.pallas_call(
        _tc_body,
        out_shape=jax.ShapeDtypeStruct((1, 32), jnp.float32),
    )(emb_dense, w_pad.reshape(1, -1), s.reshape(1, -1), t.reshape(1, -1),
      W_self1, W_neigh1, b1.reshape(1, -1), W_self2, W_neigh2, b2.reshape(1, -1))
    return out


# speedup vs baseline: 1.0062x; 1.0001x over previous
import functools

import jax
import jax.numpy as jnp
from jax import lax
from jax.experimental import pallas as pl
from jax.experimental.pallas import tpu as pltpu
from jax.experimental.pallas import tpu_sc as plsc

_NT = 16
_GC = 64


def _sc_stage(ei, ew, w_pad, attr_pad, table):
    npad = w_pad.shape[0]
    n_edges = ei.shape[1]
    nchunks_e = n_edges // 128
    base_ch = nchunks_e // _NT
    n_extra = nchunks_e - base_ch * _NT
    ept = (base_ch + 1) * 128
    rpt = npad // _NT
    f32 = jnp.float32

    mesh = plsc.VectorSubcoreMesh(core_axis_name="c", subcore_axis_name="s")

    def body(ei_hbm, ew_hbm, w_hbm, attr_hbm, table_hbm,
             s_hbm, t_hbm, emb_hbm,
             eb2_v, wb_v, g_v, acc_v, red_v,
             wc_v, cc_v, rc_v, gc_v,
             idx_v, rows_v, rows2_v,
             parts_sh, gfull_sh, zeros_sh,
             sem_g0, sem_g1, sem_o0, sem_o1, sem_e0, sem_e1):
        cid = lax.axis_index("c")
        tid = lax.axis_index("s")

        @pl.when(cid == 0)
        def _edge_passes():
            main_e = base_ch * 128
            ebase = tid * main_e + 128 * jnp.minimum(tid, n_extra)
            has_extra = tid < n_extra
            nbase = tid * rpt
            pltpu.sync_copy(ei_hbm.at[:, pl.ds(ebase, main_e)],
                            eb2_v.at[:, pl.ds(0, main_e)])
            h_ew = pltpu.async_copy(ew_hbm.at[pl.ds(ebase, main_e)],
                                    wb_v.at[pl.ds(0, main_e)], sem_e1)

            @pl.when(has_extra)
            def _extra_loads():
                pltpu.sync_copy(ei_hbm.at[:, pl.ds(ebase + main_e, 128)],
                                eb2_v.at[:, pl.ds(main_e, 128)])
                pltpu.sync_copy(ew_hbm.at[pl.ds(ebase + main_e, 128)],
                                wb_v.at[pl.ds(main_e, 128)])
            pltpu.sync_copy(w_hbm.at[pl.ds(nbase, rpt)], wc_v)

            zero16 = jnp.zeros((16,), f32)
            ones16 = jnp.ones((16,), f32)

            def zero_acc():
                def zb(i, c):
                    acc_v[pl.ds(i * 16, 16)] = zero16
                    return c
                lax.fori_loop(0, npad // 16, zb, 0)

            def reduce_to(chunk_ref):
                pltpu.sync_copy(acc_v, parts_sh.at[tid])
                plsc.subcore_barrier()
                pltpu.sync_copy(parts_sh.at[:, pl.ds(nbase, rpt)], red_v)
                plsc.subcore_barrier()

                @plsc.parallel_loop(0, rpt // 32)
                def _rb(j):
                    sl0 = pl.ds(j * 32, 16)
                    sl1 = pl.ds(j * 32 + 16, 16)
                    v0 = red_v[0, sl0]
                    v1 = red_v[0, sl1]
                    for k in range(1, _NT):
                        v0 = v0 + red_v[k, sl0]
                        v1 = v1 + red_v[k, sl1]
                    chunk_ref[sl0] = v0
                    chunk_ref[sl1] = v1

            def share_g(num_ref):
                def gb(j, c):
                    sl = pl.ds(j * 16, 16)
                    gc_v[sl] = num_ref[sl] / jnp.maximum(cc_v[sl], 1.0)
                    return c
                lax.fori_loop(0, rpt // 16, gb, 0)
                pltpu.sync_copy(gc_v, gfull_sh.at[pl.ds(nbase, rpt)])
                plsc.subcore_barrier()
                pltpu.sync_copy(gfull_sh, g_v)

            zero_acc()

            @pl.when(tid == 0)
            def _stash_zeros():
                pltpu.sync_copy(acc_v, zeros_sh)

            unroll = 8

            def p0(i, c):
                sls = [pl.ds((i * unroll + u) * 16, 16) for u in range(unroll)]
                ds_ = [eb2_v[1, sl] for sl in sls]
                for d in ds_:
                    plsc.addupdate_scatter(acc_v, [d], ones16)
                return c
            @plsc.parallel_loop(0, base_ch)
            def _p0_loop(i):
                p0(i, 0)

            @pl.when(has_extra)
            def _p0_extra():
                p0(base_ch, 0)
            h_ew.wait()
            reduce_to(cc_v)
            share_g(wc_v)

            def edge_pass(i, c):
                sls = [pl.ds((i * unroll + u) * 16, 16) for u in range(unroll)]
                ds_ = [eb2_v[1, sl] for sl in sls]
                gs_ = [plsc.load_gather(g_v, [d]) for d in ds_]
                es_ = [wb_v[sl] for sl in sls]
                ss_ = [eb2_v[0, sl] for sl in sls]
                vals = [g * e for g, e in zip(gs_, es_)]
                for s_, v in zip(ss_, vals):
                    plsc.addupdate_scatter(acc_v, [s_], v)
                return c

            def run_edge_pass():
                @plsc.parallel_loop(0, base_ch)
                def _ep(i):
                    edge_pass(i, 0)

                @pl.when(has_extra)
                def _extra():
                    edge_pass(base_ch, 0)

            pltpu.sync_copy(zeros_sh, acc_v)
            run_edge_pass()
            reduce_to(rc_v)
            pltpu.sync_copy(rc_v, s_hbm.at[pl.ds(nbase, rpt)])
            share_g(rc_v)

            pltpu.sync_copy(zeros_sh, acc_v)
            run_edge_pass()
            reduce_to(rc_v)
            pltpu.sync_copy(rc_v, t_hbm.at[pl.ds(nbase, rpt)])

        @pl.when(cid == 1)
        def _emb_gather():
            r0 = tid * rpt
            nchunks = rpt // _GC
            pltpu.sync_copy(attr_hbm.at[pl.ds(r0, rpt)], idx_v)
            rowsb = [rows_v, rows2_v]
            gsem = [sem_g0, sem_g1]
            osem = [sem_o0, sem_o1]
            gh = [None, None]
            oh = [None, None]
            gh[0] = pltpu.async_copy(
                table_hbm.at[idx_v.at[pl.ds(0, _GC)]], rows_v, sem_g0)
            for j in range(nchunks):
                p = j % 2
                q = 1 - p
                if j + 1 < nchunks:
                    if oh[q] is not None:
                        oh[q].wait()
                    gh[q] = pltpu.async_copy(
                        table_hbm.at[idx_v.at[pl.ds((j + 1) * _GC, _GC)]],
                        rowsb[q], gsem[q])
                gh[p].wait()
                oh[p] = pltpu.async_copy(
                    rowsb[p], emb_hbm.at[pl.ds(r0 + j * _GC, _GC), :], osem[p])
            for p in range(2):
                if oh[p] is not None:
                    oh[p].wait()

    call = pl.kernel(
        body,
        mesh=mesh,
        compiler_params=pltpu.CompilerParams(needs_layout_passes=False),
        out_type=[
            jax.ShapeDtypeStruct((npad,), f32),
            jax.ShapeDtypeStruct((npad,), f32),
            jax.ShapeDtypeStruct((npad, 128), f32),
        ],
        scratch_types=[
            pltpu.VMEM((2, ept), jnp.int32),
            pltpu.VMEM((ept,), f32),
            pltpu.VMEM((npad,), f32),
            pltpu.VMEM((npad,), f32),
            pltpu.VMEM((_NT, rpt), f32),
            pltpu.VMEM((rpt,), f32),
            pltpu.VMEM((rpt,), f32),
            pltpu.VMEM((rpt,), f32),
            pltpu.VMEM((rpt,), f32),
            pltpu.VMEM((rpt,), jnp.int32),
            pltpu.VMEM((_GC, 128), f32),
            pltpu.VMEM((_GC, 128), f32),
            pltpu.VMEM_SHARED((_NT, npad), f32),
            pltpu.VMEM_SHARED((npad,), f32),
            pltpu.VMEM_SHARED((npad,), f32),
            pltpu.SemaphoreType.DMA,
            pltpu.SemaphoreType.DMA,
            pltpu.SemaphoreType.DMA,
            pltpu.SemaphoreType.DMA,
            pltpu.SemaphoreType.DMA,
            pltpu.SemaphoreType.DMA,
        ],
    )
    return call(ei, ew, w_pad, attr_pad, table)


def _tc_body(emb_ref, w_ref, s_ref, t_ref, ws1_ref, wn1_ref, b1_ref,
             ws2_ref, wn2_ref, b2_ref, out_ref):
    hi = jax.lax.Precision.HIGHEST

    def dot(a, b):
        return lax.dot_general(a, b, (((1,), (0,)), ((), ())),
                               precision=hi, preferred_element_type=jnp.float32)

    emb = emb_ref[:, :]
    ssq = jnp.sum(emb * emb, axis=1, keepdims=True)
    scale = jnp.minimum(1.0, lax.rsqrt(jnp.maximum(ssq, 1e-30)))
    m = emb * scale
    w = w_ref[:, :]
    s = s_ref[:, :]
    t = t_ref[:, :]
    q = jnp.concatenate([w * w, s * w, t * w], axis=0)
    abc = dot(q, m)
    a_, b_, c_ = abc[0:1], abc[1:2], abc[2:3]
    sw = jnp.sum(w)
    ss = jnp.sum(s)
    b1 = b1_ref[:, :]
    b2 = b2_ref[:, :]
    u = dot(a_, ws1_ref[:, :]) + sw * b1 + dot(b_, wn1_ref[:, :])
    v = dot(b_, ws1_ref[:, :]) + ss * b1 + dot(c_, wn1_ref[:, :])
    out_ref[:, :] = (dot(u, ws2_ref[:, :]) + sw * b2 + dot(v, wn2_ref[:, :])) / sw


def kernel(node_attr, node_w, edge_index, edge_w, embed_table,
           W_self1, W_neigh1, b1, W_self2, W_neigh2, b2):
    n = node_attr.shape[0]
    npad = ((n + 16 * _GC - 1) // (16 * _GC)) * (16 * _GC)
    pad = npad - n

    attr = node_attr.astype(jnp.int32)
    attr_pad = jnp.concatenate([attr, jnp.zeros((pad,), jnp.int32)])
    w = node_w.reshape(-1).astype(jnp.float32)
    w_pad = jnp.concatenate([w, jnp.zeros((pad,), jnp.float32)])
    ei = edge_index.astype(jnp.int32)
    ew = edge_w.astype(jnp.float32)

    s, t, emb_dense = _sc_stage(ei, ew, w_pad, attr_pad, embed_table)

    out = pl.pallas_call(
        _tc_body,
        out_shape=jax.ShapeDtypeStruct((1, 32), jnp.float32),
    )(emb_dense, w_pad.reshape(1, -1), s.reshape(1, -1), t.reshape(1, -1),
      W_self1, W_neigh1, b1.reshape(1, -1), W_self2, W_neigh2, b2.reshape(1, -1))
    return out


# speedup vs baseline: 1.0118x; 1.0056x over previous
import jax
import jax.numpy as jnp
from jax import lax
from jax.experimental import pallas as pl
from jax.experimental.pallas import tpu as pltpu
from jax.experimental.pallas import tpu_sc as plsc

_NT = 16
_GC = 64


def _sc_stage(ei, ew, w_pad, attr_pad, table):
    npad = w_pad.shape[0]
    n_edges = ei.shape[1]
    nchunks_e = n_edges // 128
    base_ch = nchunks_e // _NT
    n_extra = nchunks_e - base_ch * _NT
    ept = (base_ch + 1) * 128
    rpt = npad // _NT
    f32 = jnp.float32

    mesh = plsc.VectorSubcoreMesh(core_axis_name="c", subcore_axis_name="s")

    def body(ei_hbm, ew_hbm, w_hbm, attr_hbm, table_hbm,
             s_hbm, t_hbm, emb_hbm,
             eb2_v, wb_v, g_v, acc_v, red_v,
             wc_v, cc_v, rc_v, gc_v,
             idx_v, rows_v, rows2_v,
             parts_sh, gfull_sh, zeros_sh,
             sem_g0, sem_g1, sem_o0, sem_o1, sem_e0, sem_e1):
        cid = lax.axis_index("c")
        tid = lax.axis_index("s")

        @pl.when(cid == 0)
        def _edge_passes():
            main_e = base_ch * 128
            ebase = tid * main_e + 128 * jnp.minimum(tid, n_extra)
            has_extra = tid < n_extra
            nbase = tid * rpt
            pltpu.sync_copy(ei_hbm.at[:, pl.ds(ebase, main_e)],
                            eb2_v.at[:, pl.ds(0, main_e)])
            h_ew = pltpu.async_copy(ew_hbm.at[pl.ds(ebase, main_e)],
                                    wb_v.at[pl.ds(0, main_e)], sem_e1)

            @pl.when(has_extra)
            def _extra_loads():
                pltpu.sync_copy(ei_hbm.at[:, pl.ds(ebase + main_e, 128)],
                                eb2_v.at[:, pl.ds(main_e, 128)])
                pltpu.sync_copy(ew_hbm.at[pl.ds(ebase + main_e, 128)],
                                wb_v.at[pl.ds(main_e, 128)])
            pltpu.sync_copy(w_hbm.at[pl.ds(nbase, rpt)], wc_v)

            zero16 = jnp.zeros((16,), f32)
            ones16 = jnp.ones((16,), f32)

            def zero_acc():
                def zb(i, c):
                    acc_v[pl.ds(i * 16, 16)] = zero16
                    return c
                lax.fori_loop(0, npad // 16, zb, 0)

            def reduce_to(chunk_ref):
                pltpu.sync_copy(acc_v, parts_sh.at[tid])
                plsc.subcore_barrier()
                pltpu.sync_copy(parts_sh.at[:, pl.ds(nbase, rpt)], red_v)
                plsc.subcore_barrier()

                @plsc.parallel_loop(0, rpt // 32)
                def _rb(j):
                    sl0 = pl.ds(j * 32, 16)
                    sl1 = pl.ds(j * 32 + 16, 16)
                    v0 = red_v[0, sl0]
                    v1 = red_v[0, sl1]
                    for k in range(1, _NT):
                        v0 = v0 + red_v[k, sl0]
                        v1 = v1 + red_v[k, sl1]
                    chunk_ref[sl0] = v0
                    chunk_ref[sl1] = v1

            def share_g(num_ref):
                def gb(j, c):
                    sl = pl.ds(j * 16, 16)
                    gc_v[sl] = num_ref[sl] / jnp.maximum(cc_v[sl], 1.0)
                    return c
                lax.fori_loop(0, rpt // 16, gb, 0)
                pltpu.sync_copy(gc_v, gfull_sh.at[pl.ds(nbase, rpt)])
                plsc.subcore_barrier()
                pltpu.sync_copy(gfull_sh, g_v)

            zero_acc()

            @pl.when(tid == 0)
            def _stash_zeros():
                pltpu.sync_copy(acc_v, zeros_sh)

            unroll = 8

            def p0(i, c):
                sls = [pl.ds((i * unroll + u) * 16, 16) for u in range(unroll)]
                ds_ = [eb2_v[1, sl] for sl in sls]
                for d in ds_:
                    plsc.addupdate_scatter(acc_v, [d], ones16)
                return c
            @plsc.parallel_loop(0, base_ch)
            def _p0_loop(i):
                p0(i, 0)

            @pl.when(has_extra)
            def _p0_extra():
                p0(base_ch, 0)
            h_ew.wait()
            reduce_to(cc_v)
            share_g(wc_v)

            def edge_pass(i, c):
                sls = [pl.ds((i * unroll + u) * 16, 16) for u in range(unroll)]
                ds_ = [eb2_v[1, sl] for sl in sls]
                gs_ = [plsc.load_gather(g_v, [d]) for d in ds_]
                es_ = [wb_v[sl] for sl in sls]
                ss_ = [eb2_v[0, sl] for sl in sls]
                vals = [g * e for g, e in zip(gs_, es_)]
                for s_, v in zip(ss_, vals):
                    plsc.addupdate_scatter(acc_v, [s_], v)
                return c

            def run_edge_pass():
                @plsc.parallel_loop(0, base_ch)
                def _ep(i):
                    edge_pass(i, 0)

                @pl.when(has_extra)
                def _extra():
                    edge_pass(base_ch, 0)

            pltpu.sync_copy(zeros_sh, acc_v)
            run_edge_pass()
            reduce_to(rc_v)
            pltpu.sync_copy(rc_v, s_hbm.at[pl.ds(nbase, rpt)])
            share_g(rc_v)

            pltpu.sync_copy(zeros_sh, acc_v)
            run_edge_pass()
            reduce_to(rc_v)
            pltpu.sync_copy(rc_v, t_hbm.at[pl.ds(nbase, rpt)])

        @pl.when(cid == 1)
        def _emb_gather():
            r0 = tid * rpt
            nchunks = rpt // _GC
            pltpu.sync_copy(attr_hbm.at[pl.ds(r0, rpt)], idx_v)
            rowsb = [rows_v, rows2_v]
            gsem = [sem_g0, sem_g1]
            osem = [sem_o0, sem_o1]
            gh = [None, None]
            oh = [None, None]
            gh[0] = pltpu.async_copy(
                table_hbm.at[idx_v.at[pl.ds(0, _GC)]], rows_v, sem_g0)
            for j in range(nchunks):
                p = j % 2
                q = 1 - p
                if j + 1 < nchunks:
                    if oh[q] is not None:
                        oh[q].wait()
                    gh[q] = pltpu.async_copy(
                        table_hbm.at[idx_v.at[pl.ds((j + 1) * _GC, _GC)]],
                        rowsb[q], gsem[q])
                gh[p].wait()
                oh[p] = pltpu.async_copy(
                    rowsb[p], emb_hbm.at[pl.ds(r0 + j * _GC, _GC), :], osem[p])
            for p in range(2):
                if oh[p] is not None:
                    oh[p].wait()

    call = pl.kernel(
        body,
        mesh=mesh,
        compiler_params=pltpu.CompilerParams(needs_layout_passes=False),
        out_type=[
            jax.ShapeDtypeStruct((npad,), f32),
            jax.ShapeDtypeStruct((npad,), f32),
            jax.ShapeDtypeStruct((npad, 128), f32),
        ],
        scratch_types=[
            pltpu.VMEM((2, ept), jnp.int32),
            pltpu.VMEM((ept,), f32),
            pltpu.VMEM((npad,), f32),
            pltpu.VMEM((npad,), f32),
            pltpu.VMEM((_NT, rpt), f32),
            pltpu.VMEM((rpt,), f32),
            pltpu.VMEM((rpt,), f32),
            pltpu.VMEM((rpt,), f32),
            pltpu.VMEM((rpt,), f32),
            pltpu.VMEM((rpt,), jnp.int32),
            pltpu.VMEM((_GC, 128), f32),
            pltpu.VMEM((_GC, 128), f32),
            pltpu.VMEM_SHARED((_NT, npad), f32),
            pltpu.VMEM_SHARED((npad,), f32),
            pltpu.VMEM_SHARED((npad,), f32),
            pltpu.SemaphoreType.DMA,
            pltpu.SemaphoreType.DMA,
            pltpu.SemaphoreType.DMA,
            pltpu.SemaphoreType.DMA,
            pltpu.SemaphoreType.DMA,
            pltpu.SemaphoreType.DMA,
        ],
    )
    return call(ei, ew, w_pad, attr_pad, table)


def _tc_body(emb_ref, w_ref, s_ref, t_ref, ws1_ref, wn1_ref, b1_ref,
             ws2_ref, wn2_ref, b2_ref, out_ref):
    hi = jax.lax.Precision.HIGHEST

    def dot(a, b):
        return lax.dot_general(a, b, (((1,), (0,)), ((), ())),
                               precision=hi, preferred_element_type=jnp.float32)

    emb = emb_ref[:, :]
    ssq = jnp.sum(emb * emb, axis=1, keepdims=True)
    scale = jnp.minimum(1.0, lax.rsqrt(jnp.maximum(ssq, 1e-30)))
    m = emb * scale
    w = w_ref[:, :]
    s = s_ref[:, :]
    t = t_ref[:, :]
    q = jnp.concatenate([w * w, s * w, t * w], axis=0)
    abc = dot(q, m)
    a_, b_, c_ = abc[0:1], abc[1:2], abc[2:3]
    sw = jnp.sum(w)
    ss = jnp.sum(s)
    b1 = b1_ref[:, :]
    b2 = b2_ref[:, :]
    u = dot(a_, ws1_ref[:, :]) + sw * b1 + dot(b_, wn1_ref[:, :])
    v = dot(b_, ws1_ref[:, :]) + ss * b1 + dot(c_, wn1_ref[:, :])
    out_ref[:, :] = (dot(u, ws2_ref[:, :]) + sw * b2 + dot(v, wn2_ref[:, :])) / sw


def kernel(node_attr, node_w, edge_index, edge_w, embed_table,
           W_self1, W_neigh1, b1, W_self2, W_neigh2, b2):
    n = node_attr.shape[0]
    npad = ((n + 16 * _GC - 1) // (16 * _GC)) * (16 * _GC)
    pad = npad - n

    attr = node_attr.astype(jnp.int32)
    attr_pad = jnp.concatenate([attr, jnp.zeros((pad,), jnp.int32)])
    w = node_w.reshape(-1).astype(jnp.float32)
    w_pad = jnp.concatenate([w, jnp.zeros((pad,), jnp.float32)])
    ei = edge_index.astype(jnp.int32)
    ew = edge_w.astype(jnp.float32)

    s, t, emb_dense = _sc_stage(ei, ew, w_pad, attr_pad, embed_table)

    out = pl.pallas_call(
        _tc_body,
        out_shape=jax.ShapeDtypeStruct((1, 32), jnp.float32),
    )(emb_dense, w_pad.reshape(1, -1), s.reshape(1, -1), t.reshape(1, -1),
      W_self1, W_neigh1, b1.reshape(1, -1), W_self2, W_neigh2, b2.reshape(1, -1))
    return out
